# baseline (speedup 1.0000x reference)
import jax
import jax.numpy as jnp
from jax.experimental import pallas as pl
from jax.experimental.pallas import tpu as pltpu

B = 16
H = 16
N = 512
D = 64
HIDDEN = 256
D_MODEL = H * D
M = N // 2
L = 2 * D


def _body(k_ref, v_ref, h_ref, wkv_ref, wh_ref, b1_ref, wf_ref, b2_ref,
          outk_ref, outv_ref, outs_ref, acc_ref):
    hd = pl.program_id(1)

    outk_ref[...] = k_ref[...]
    outv_ref[...] = v_ref[...]

    @pl.when(hd == 0)
    def _init():
        acc_ref[...] = (
            jnp.dot(h_ref[0], wh_ref[...], preferred_element_type=jnp.float32)
            + b1_ref[...]
        )

    kv = jnp.concatenate([k_ref[0, 0], v_ref[0, 0]], axis=-1)
    acc_ref[...] += jnp.dot(kv, wkv_ref[hd],
                            preferred_element_type=jnp.float32)

    @pl.when(hd == H - 1)
    def _finish():
        a = jnp.maximum(acc_ref[...], 0.0)
        outs_ref[0] = (
            jnp.dot(a, wf_ref[...], preferred_element_type=jnp.float32)
            + b2_ref[0, 0]
        )


def kernel(key_t, value_t, hidden_state, w1, b1, w2, b2, t1_k, t1_v, t1_scores):
    k2 = key_t.reshape(B, H, M, L)
    v2 = value_t.reshape(B, H, M, L)
    h2 = hidden_state.reshape(B, M, 2 * D_MODEL)

    eye2 = jnp.eye(2, dtype=jnp.float32)
    wk = w1[:D_MODEL].reshape(H, D, HIDDEN)
    wv = w1[D_MODEL:2 * D_MODEL].reshape(H, D, HIDDEN)
    wh = w1[2 * D_MODEL:]
    wk2 = (eye2[None, :, None, :, None]
           * wk[:, None, :, None, :]).reshape(H, L, 2 * HIDDEN)
    wv2 = (eye2[None, :, None, :, None]
           * wv[:, None, :, None, :]).reshape(H, L, 2 * HIDDEN)
    wkv2 = jnp.concatenate([wk2, wv2], axis=1)
    wh2 = (eye2[:, None, :, None]
           * wh[None, :, None, :]).reshape(2 * D_MODEL, 2 * HIDDEN)
    b1_2 = jnp.tile(b1, 2).reshape(1, 2 * HIDDEN)
    wf = (eye2[:, None, :] * w2[:, 0][None, :, None]).reshape(2 * HIDDEN, 2)
    b2r = b2.reshape(1, 1)

    grid = (B, H)
    out_shape = (
        jax.ShapeDtypeStruct((B, H, M, L), jnp.float32),
        jax.ShapeDtypeStruct((B, H, M, L), jnp.float32),
        jax.ShapeDtypeStruct((B, M, 2), jnp.float32),
    )
    outk, outv, outs = pl.pallas_call(
        _body,
        grid=grid,
        in_specs=[
            pl.BlockSpec((1, 1, M, L), lambda b, h: (b, h, 0, 0)),
            pl.BlockSpec((1, 1, M, L), lambda b, h: (b, h, 0, 0)),
            pl.BlockSpec((1, M, 2 * D_MODEL), lambda b, h: (b, 0, 0)),
            pl.BlockSpec((H, L + L, 2 * HIDDEN), lambda b, h: (0, 0, 0)),
            pl.BlockSpec((2 * D_MODEL, 2 * HIDDEN), lambda b, h: (0, 0)),
            pl.BlockSpec((1, 2 * HIDDEN), lambda b, h: (0, 0)),
            pl.BlockSpec((2 * HIDDEN, 2), lambda b, h: (0, 0)),
            pl.BlockSpec((1, 1), lambda b, h: (0, 0)),
        ],
        out_specs=[
            pl.BlockSpec((1, 1, M, L), lambda b, h: (b, h, 0, 0)),
            pl.BlockSpec((1, 1, M, L), lambda b, h: (b, h, 0, 0)),
            pl.BlockSpec((1, M, 2), lambda b, h: (b, 0, 0)),
        ],
        out_shape=out_shape,
        scratch_shapes=[pltpu.VMEM((M, 2 * HIDDEN), jnp.float32)],
    )(k2, v2, h2, wkv2, wh2, b1_2, wf, b2r)
    return (outk.reshape(B, H, N, D), outv.reshape(B, H, N, D),
            outs.reshape(B, N))


# speedup vs baseline: 1.3440x; 1.3440x over previous
import jax
import jax.numpy as jnp
from jax.experimental import pallas as pl

B = 16
H = 16
N = 512
D = 64
HIDDEN = 256
D_MODEL = H * D
M = N // 2
L = 2 * D


def _body(k_ref, v_ref, h_ref, wkv_ref, wh_ref, b1_ref, wf_ref, b2_ref,
          outk_ref, outv_ref, outs_ref):
    outk_ref[...] = k_ref[...]
    outv_ref[...] = v_ref[...]

    acc = (jnp.dot(h_ref[0], wh_ref[...], preferred_element_type=jnp.float32)
           + b1_ref[...])
    for hd in range(H):
        kv = jnp.concatenate([k_ref[0, hd], v_ref[0, hd]], axis=-1)
        acc += jnp.dot(kv, wkv_ref[hd], preferred_element_type=jnp.float32)
    a = jnp.maximum(acc, 0.0)
    outs_ref[0] = (jnp.dot(a, wf_ref[...], preferred_element_type=jnp.float32)
                   + b2_ref[0, 0])


def kernel(key_t, value_t, hidden_state, w1, b1, w2, b2, t1_k, t1_v, t1_scores):
    k2 = key_t.reshape(B, H, M, L)
    v2 = value_t.reshape(B, H, M, L)
    h2 = hidden_state.reshape(B, M, 2 * D_MODEL)

    eye2 = jnp.eye(2, dtype=jnp.float32)
    wk = w1[:D_MODEL].reshape(H, D, HIDDEN)
    wv = w1[D_MODEL:2 * D_MODEL].reshape(H, D, HIDDEN)
    wh = w1[2 * D_MODEL:]
    wk2 = (eye2[None, :, None, :, None]
           * wk[:, None, :, None, :]).reshape(H, L, 2 * HIDDEN)
    wv2 = (eye2[None, :, None, :, None]
           * wv[:, None, :, None, :]).reshape(H, L, 2 * HIDDEN)
    wkv2 = jnp.concatenate([wk2, wv2], axis=1)
    wh2 = (eye2[:, None, :, None]
           * wh[None, :, None, :]).reshape(2 * D_MODEL, 2 * HIDDEN)
    b1_2 = jnp.tile(b1, 2).reshape(1, 2 * HIDDEN)
    wf = (eye2[:, None, :] * w2[:, 0][None, :, None]).reshape(2 * HIDDEN, 2)
    b2r = b2.reshape(1, 1)

    grid = (B,)
    out_shape = (
        jax.ShapeDtypeStruct((B, H, M, L), jnp.float32),
        jax.ShapeDtypeStruct((B, H, M, L), jnp.float32),
        jax.ShapeDtypeStruct((B, M, 2), jnp.float32),
    )
    outk, outv, outs = pl.pallas_call(
        _body,
        grid=grid,
        in_specs=[
            pl.BlockSpec((1, H, M, L), lambda b: (b, 0, 0, 0)),
            pl.BlockSpec((1, H, M, L), lambda b: (b, 0, 0, 0)),
            pl.BlockSpec((1, M, 2 * D_MODEL), lambda b: (b, 0, 0)),
            pl.BlockSpec((H, 2 * L, 2 * HIDDEN), lambda b: (0, 0, 0)),
            pl.BlockSpec((2 * D_MODEL, 2 * HIDDEN), lambda b: (0, 0)),
            pl.BlockSpec((1, 2 * HIDDEN), lambda b: (0, 0)),
            pl.BlockSpec((2 * HIDDEN, 2), lambda b: (0, 0)),
            pl.BlockSpec((1, 1), lambda b: (0, 0)),
        ],
        out_specs=[
            pl.BlockSpec((1, H, M, L), lambda b: (b, 0, 0, 0)),
            pl.BlockSpec((1, H, M, L), lambda b: (b, 0, 0, 0)),
            pl.BlockSpec((1, M, 2), lambda b: (b, 0, 0)),
        ],
        out_shape=out_shape,
    )(k2, v2, h2, wkv2, wh2, b1_2, wf, b2r)
    return (outk.reshape(B, H, N, D), outv.reshape(B, H, N, D),
            outs.reshape(B, N))


# speedup vs baseline: 1.7306x; 1.2876x over previous
import jax
import jax.numpy as jnp
from jax.experimental import pallas as pl
from jax.experimental.pallas import tpu as pltpu

B = 16
H = 16
N = 512
D = 64
HIDDEN = 256
D_MODEL = H * D
G = 4
NG = H // G


def _body(k_ref, v_ref, h_ref, wk_ref, wv_ref, wh_ref, b1_ref, w2_ref,
          b2_ref, outk_ref, outv_ref, outs_ref, acc_ref):
    b = pl.program_id(0)
    g = pl.program_id(1)

    outk_ref[...] = k_ref[...]
    outv_ref[...] = v_ref[...]

    @pl.when(g == 0)
    def _init():
        acc_ref[...] = (
            jnp.dot(h_ref[0], wh_ref[...], preferred_element_type=jnp.float32)
            + b1_ref[...]
        )

    contrib = jnp.dot(k_ref[0, 0], wk_ref[0],
                      preferred_element_type=jnp.float32)
    contrib += jnp.dot(v_ref[0, 0], wv_ref[0],
                       preferred_element_type=jnp.float32)
    for j in range(1, G):
        contrib += jnp.dot(k_ref[0, j], wk_ref[j],
                           preferred_element_type=jnp.float32)
        contrib += jnp.dot(v_ref[0, j], wv_ref[j],
                           preferred_element_type=jnp.float32)
    acc_ref[...] += contrib

    @pl.when(g == NG - 1)
    def _finish():
        a = jnp.maximum(acc_ref[...], 0.0)
        s = jnp.sum(a * w2_ref[...], axis=1) + b2_ref[0, 0]
        outs_ref[pl.ds(b, 1), :] = s[None, :]


def kernel(key_t, value_t, hidden_state, w1, b1, w2, b2, t1_k, t1_v, t1_scores):
    wk = w1[:D_MODEL].reshape(H, D, HIDDEN)
    wv = w1[D_MODEL:2 * D_MODEL].reshape(H, D, HIDDEN)
    wh = w1[2 * D_MODEL:]
    b1r = b1.reshape(1, HIDDEN)
    w2r = w2.reshape(1, HIDDEN)
    b2r = b2.reshape(1, 1)

    grid = (B, NG)
    out_shape = (
        jax.ShapeDtypeStruct((B, H, N, D), jnp.float32),
        jax.ShapeDtypeStruct((B, H, N, D), jnp.float32),
        jax.ShapeDtypeStruct((B, N), jnp.float32),
    )
    outk, outv, outs = pl.pallas_call(
        _body,
        grid=grid,
        in_specs=[
            pl.BlockSpec((1, G, N, D), lambda b, g: (b, g, 0, 0)),
            pl.BlockSpec((1, G, N, D), lambda b, g: (b, g, 0, 0)),
            pl.BlockSpec((1, N, D_MODEL), lambda b, g: (b, 0, 0)),
            pl.BlockSpec((G, D, HIDDEN), lambda b, g: (g, 0, 0)),
            pl.BlockSpec((G, D, HIDDEN), lambda b, g: (g, 0, 0)),
            pl.BlockSpec((D_MODEL, HIDDEN), lambda b, g: (0, 0)),
            pl.BlockSpec((1, HIDDEN), lambda b, g: (0, 0)),
            pl.BlockSpec((1, HIDDEN), lambda b, g: (0, 0)),
            pl.BlockSpec((1, 1), lambda b, g: (0, 0)),
        ],
        out_specs=[
            pl.BlockSpec((1, G, N, D), lambda b, g: (b, g, 0, 0)),
            pl.BlockSpec((1, G, N, D), lambda b, g: (b, g, 0, 0)),
            pl.BlockSpec((B, N), lambda b, g: (0, 0)),
        ],
        out_shape=out_shape,
        scratch_shapes=[pltpu.VMEM((N, HIDDEN), jnp.float32)],
    )(key_t, value_t, hidden_state, wk, wv, wh, b1r, w2r, b2r)
    return (outk, outv, outs)


# speedup vs baseline: 8.8208x; 5.0971x over previous
import jax
import jax.numpy as jnp
from jax.experimental import pallas as pl

B = 16
H = 16
N = 512
D = 64
HIDDEN = 256
D_MODEL = H * D

_NT = (((0,), (0,)), ((), ()))
_TT = (((0,), (1,)), ((), ()))


def _body(k_ref, v_ref, h_ref, wk_ref, wv_ref, wh_ref, b1_ref, w2_ref,
          b2_ref, outk_ref, outv_ref, outs_ref):
    b = pl.program_id(0)

    outk_ref[...] = k_ref[...]
    outv_ref[...] = v_ref[...]

    xk = k_ref[0].reshape(D_MODEL, N)
    xv = v_ref[0].reshape(D_MODEL, N)
    acc = jax.lax.dot_general(wk_ref[...], xk, _NT,
                              preferred_element_type=jnp.float32)
    acc += jax.lax.dot_general(wv_ref[...], xv, _NT,
                               preferred_element_type=jnp.float32)
    acc += jax.lax.dot_general(wh_ref[...], h_ref[0], _TT,
                               preferred_element_type=jnp.float32)
    acc += b1_ref[...]
    a = jnp.maximum(acc, 0.0)
    s = (jnp.dot(w2_ref[...], a, preferred_element_type=jnp.float32)
         + b2_ref[0, 0])
    outs_ref[pl.ds(b, 1), :] = s


def kernel(key_t, value_t, hidden_state, w1, b1, w2, b2, t1_k, t1_v, t1_scores):
    kt = jnp.swapaxes(key_t, 2, 3)
    vt = jnp.swapaxes(value_t, 2, 3)

    wk = w1[:D_MODEL]
    wv = w1[D_MODEL:2 * D_MODEL]
    wh = w1[2 * D_MODEL:]
    b1c = b1.reshape(HIDDEN, 1)
    w2r = w2.reshape(1, HIDDEN)
    b2r = b2.reshape(1, 1)

    grid = (B,)
    out_shape = (
        jax.ShapeDtypeStruct((B, H, D, N), jnp.float32),
        jax.ShapeDtypeStruct((B, H, D, N), jnp.float32),
        jax.ShapeDtypeStruct((B, N), jnp.float32),
    )
    outk, outv, outs = pl.pallas_call(
        _body,
        grid=grid,
        in_specs=[
            pl.BlockSpec((1, H, D, N), lambda b: (b, 0, 0, 0)),
            pl.BlockSpec((1, H, D, N), lambda b: (b, 0, 0, 0)),
            pl.BlockSpec((1, N, D_MODEL), lambda b: (b, 0, 0)),
            pl.BlockSpec((D_MODEL, HIDDEN), lambda b: (0, 0)),
            pl.BlockSpec((D_MODEL, HIDDEN), lambda b: (0, 0)),
            pl.BlockSpec((D_MODEL, HIDDEN), lambda b: (0, 0)),
            pl.BlockSpec((HIDDEN, 1), lambda b: (0, 0)),
            pl.BlockSpec((1, HIDDEN), lambda b: (0, 0)),
            pl.BlockSpec((1, 1), lambda b: (0, 0)),
        ],
        out_specs=[
            pl.BlockSpec((1, H, D, N), lambda b: (b, 0, 0, 0)),
            pl.BlockSpec((1, H, D, N), lambda b: (b, 0, 0, 0)),
            pl.BlockSpec((B, N), lambda b: (0, 0)),
        ],
        out_shape=out_shape,
    )(kt, vt, hidden_state, wk, wv, wh, b1c, w2r, b2r)
    return (jnp.swapaxes(outk, 2, 3), jnp.swapaxes(outv, 2, 3), outs)
